# Initial kernel scaffold; baseline (speedup 1.0000x reference)
import jax
import jax.numpy as jnp
from jax.experimental import pallas as pl


def kernel(x, adj, Wl1, bl1, Wr1, Wl2, bl2, Wr2):
    raise NotImplementedError("write your pallas kernel here")


# speedup vs baseline: 5.6371x; 5.6371x over previous
import functools

import jax
import jax.numpy as jnp
from jax import lax
from jax.experimental import pallas as pl
from jax.experimental.pallas import tpu as pltpu
from jax.experimental.pallas import tpu_sc as plsc

N_NODES = 10000
N_EDGES = 320000
D = 128

NC = 2
NS = 16
NW = NC * NS
EPT = N_EDGES // NW
CH = 80
NCH = EPT // CH
NPAD = 10240
RPT = NPAD // NS
ZR = 64


def _sc_agg(with_cnt: bool):
    out_type = [jax.ShapeDtypeStruct((NC, NPAD, D), jnp.float32)]
    scratch = [
        pltpu.VMEM((CH,), jnp.int32),
        pltpu.VMEM((CH,), jnp.int32),
        pltpu.VMEM((CH, D), jnp.float32),
        pltpu.VMEM((ZR, D), jnp.float32),
        pltpu.VMEM_SHARED((NPAD, D), jnp.float32),
        pltpu.SemaphoreType.DMA,
    ]
    if with_cnt:
        out_type.append(jax.ShapeDtypeStruct((NW, NPAD), jnp.float32))
        scratch.append(pltpu.VMEM((NPAD,), jnp.float32))

    mesh = plsc.VectorSubcoreMesh(core_axis_name="c", subcore_axis_name="s")

    @functools.partial(
        pl.kernel, out_type=out_type, scratch_types=scratch, mesh=mesh,
        compiler_params=pltpu.CompilerParams(needs_layout_passes=False))
    def body(*refs):
        if with_cnt:
            (x_hbm, src_hbm, dst_hbm, zrow_hbm, zcnt_hbm,
             agg_out, cnt_out,
             srcv, dstv, rows, zbuf, acc, sem, cntv) = refs
        else:
            (x_hbm, src_hbm, dst_hbm, zrow_hbm,
             agg_out,
             srcv, dstv, rows, zbuf, acc, sem) = refs

        c = lax.axis_index("c")
        s = lax.axis_index("s")
        wid = c * NS + s

        pltpu.sync_copy(zrow_hbm, zbuf)
        for j in range(RPT // ZR):
            pltpu.sync_copy(zbuf, acc.at[pl.ds(s * RPT + j * ZR, ZR)])
        if with_cnt:
            pltpu.sync_copy(zcnt_hbm, cntv)
        plsc.subcore_barrier()

        base0 = wid * EPT
        ones16 = jnp.full((16,), 1.0, jnp.float32)

        def chunk(i, carry):
            b = pl.multiple_of(base0 + i * CH, 8)
            pltpu.sync_copy(src_hbm.at[pl.ds(b, CH)], srcv)
            pltpu.sync_copy(dst_hbm.at[pl.ds(b, CH)], dstv)
            pltpu.async_copy(x_hbm.at[srcv], rows, sem).wait()
            pltpu.sync_copy(rows, acc.at[dstv], add=True)
            if with_cnt:
                for g in range(CH // 16):
                    dst16 = dstv[pl.ds(g * 16, 16)]
                    plsc.addupdate_scatter(cntv, [dst16], ones16)
            return carry

        lax.fori_loop(0, NCH, chunk, 0)
        plsc.subcore_barrier()

        r0 = s * RPT
        pltpu.sync_copy(acc.at[pl.ds(r0, RPT)], agg_out.at[c, pl.ds(r0, RPT)])
        if with_cnt:
            pltpu.sync_copy(cntv, cnt_out.at[wid])

    return body


_sc_agg_cnt_kernel = _sc_agg(with_cnt=True)
_sc_agg_kernel = _sc_agg(with_cnt=False)

BLK = 1024


def _tc_body(agg_ref, cnt_ref, x_ref, wl_ref, bl_ref, wr_ref, o_ref):
    agg = agg_ref[0] + agg_ref[1]
    cnt = lax.dot_general(cnt_ref[...], jnp.ones((NW, D), jnp.float32),
                          (((0,), (0,)), ((), ())),
                          preferred_element_type=jnp.float32)
    mean = agg / jnp.maximum(cnt, 1.0)
    acc = jnp.dot(mean, wl_ref[...], preferred_element_type=jnp.float32)
    acc = acc + bl_ref[...]
    acc = acc + jnp.dot(x_ref[...], wr_ref[...],
                        preferred_element_type=jnp.float32)
    o_ref[...] = jnp.maximum(acc, 0.0)


def _tc_layer(aggp, cntp, x, Wl, bl2d, Wr):
    return pl.pallas_call(
        _tc_body,
        grid=((N_NODES + BLK - 1) // BLK,),
        in_specs=[
            pl.BlockSpec((NC, BLK, D), lambda i: (0, i, 0)),
            pl.BlockSpec((NW, BLK), lambda i: (0, i)),
            pl.BlockSpec((BLK, D), lambda i: (i, 0)),
            pl.BlockSpec((D, D), lambda i: (0, 0)),
            pl.BlockSpec((1, D), lambda i: (0, 0)),
            pl.BlockSpec((D, D), lambda i: (0, 0)),
        ],
        out_specs=pl.BlockSpec((BLK, D), lambda i: (i, 0)),
        out_shape=jax.ShapeDtypeStruct((N_NODES, D), jnp.float32),
    )(aggp, cntp, x, Wl, bl2d, Wr)


def kernel(x, adj, Wl1, bl1, Wr1, Wl2, bl2, Wr2):
    adj = adj.astype(jnp.int32)
    src = adj[0]
    dst = adj[1]
    zrow = jnp.zeros((ZR, D), jnp.float32)
    zcnt = jnp.zeros((NPAD,), jnp.float32)

    aggp, cntp = _sc_agg_cnt_kernel(x, src, dst, zrow, zcnt)
    h = _tc_layer(aggp, cntp, x, Wl1, bl1.reshape(1, D), Wr1)
    (aggp2,) = _sc_agg_kernel(h, src, dst, zrow)
    out = _tc_layer(aggp2, cntp, h, Wl2, bl2.reshape(1, D), Wr2)
    return out


# speedup vs baseline: 11.9383x; 2.1178x over previous
import functools

import jax
import jax.numpy as jnp
from jax import lax
from jax.experimental import pallas as pl
from jax.experimental.pallas import tpu as pltpu
from jax.experimental.pallas import tpu_sc as plsc

N_NODES = 10000
N_EDGES = 320000
D = 128

NC = 2
NS = 16
NW = NC * NS
CH = 128
NCH = 80
EPT = NCH * CH
E_PAD = NW * EPT
NPAIR = NCH // 2 - 1
NPAD = 10240
RPT = NPAD // NS


def _sc_agg(with_cnt: bool):
    out_type = [jax.ShapeDtypeStruct((NC, NPAD, D), jnp.float32)]
    scratch = [
        pltpu.VMEM((CH,), jnp.int32),
        pltpu.VMEM((CH,), jnp.int32),
        pltpu.VMEM((CH,), jnp.int32),
        pltpu.VMEM((CH,), jnp.int32),
        pltpu.VMEM((CH, D), jnp.float32),
        pltpu.VMEM((CH, D), jnp.float32),
        pltpu.VMEM_SHARED((NPAD, D), jnp.float32),
        pltpu.SemaphoreType.DMA,
        pltpu.SemaphoreType.DMA,
        pltpu.SemaphoreType.DMA,
        pltpu.SemaphoreType.DMA,
        pltpu.SemaphoreType.DMA,
        pltpu.SemaphoreType.DMA,
        pltpu.SemaphoreType.DMA,
        pltpu.SemaphoreType.DMA,
    ]
    if with_cnt:
        out_type.append(jax.ShapeDtypeStruct((NW, NPAD), jnp.float32))
        scratch.append(pltpu.VMEM((NPAD,), jnp.float32))

    mesh = plsc.VectorSubcoreMesh(core_axis_name="c", subcore_axis_name="s")

    @functools.partial(
        pl.kernel, out_type=out_type, scratch_types=scratch, mesh=mesh,
        compiler_params=pltpu.CompilerParams(needs_layout_passes=False))
    def body(*refs):
        if with_cnt:
            (x_hbm, src_hbm, dst_hbm, zrow_hbm, zcnt_hbm,
             agg_out, cnt_out,
             srcv0, srcv1, dstv0, dstv1, rows0, rows1, acc,
             isems0, isems1, isemd0, isemd1,
             gsem0, gsem1, ssem0, ssem1, cntv) = refs
        else:
            (x_hbm, src_hbm, dst_hbm, zrow_hbm,
             agg_out,
             srcv0, srcv1, dstv0, dstv1, rows0, rows1, acc,
             isems0, isems1, isemd0, isemd1,
             gsem0, gsem1, ssem0, ssem1) = refs

        c = lax.axis_index("c")
        s = lax.axis_index("s")
        wid = c * NS + s

        pltpu.sync_copy(zrow_hbm, acc.at[pl.ds(s * RPT, RPT)])
        if with_cnt:
            pltpu.sync_copy(zcnt_hbm, cntv)
        plsc.subcore_barrier()

        ones16 = jnp.full((16,), 1.0, jnp.float32)
        base0 = wid * EPT

        def fsrc(i, buf, sem):
            b = pl.multiple_of(base0 + i * CH, 8)
            return pltpu.make_async_copy(src_hbm.at[pl.ds(b, CH)], buf, sem)

        def fdst(i, buf, sem):
            b = pl.multiple_of(base0 + i * CH, 8)
            return pltpu.make_async_copy(dst_hbm.at[pl.ds(b, CH)], buf, sem)

        def gather(buf, idxbuf, sem):
            return pltpu.make_async_copy(x_hbm.at[idxbuf], buf, sem)

        def scatter(buf, idxbuf, sem):
            return pltpu.make_async_copy(buf, acc.at[idxbuf], sem)

        def count(dbuf):
            if with_cnt:
                for g in range(CH // 16):
                    dst16 = dbuf[pl.ds(g * 16, 16)]
                    plsc.addupdate_scatter(cntv, [dst16], ones16)

        fsrc(0, srcv0, isems0).start()
        fsrc(0, srcv0, isems0).wait()
        gather(rows0, srcv0, gsem0).start()
        fsrc(1, srcv1, isems1).start()
        fdst(0, dstv0, isemd0).start()

        def pair_body(i0, first=False, last=False):
            i1 = i0 + 1
            gather(rows0, srcv0, gsem0).wait()
            if not last:
                fsrc(i0 + 2, srcv0, isems0).start()
            if not first:
                scatter(rows1, dstv1, ssem1).wait()
            fdst(i1, dstv1, isemd1).start()
            fsrc(i1, srcv1, isems1).wait()
            gather(rows1, srcv1, gsem1).start()
            fdst(i0, dstv0, isemd0).wait()
            sc0 = scatter(rows0, dstv0, ssem0)
            sc0.start(add=True)
            count(dstv0)
            gather(rows1, srcv1, gsem1).wait()
            sc0.wait()
            if not last:
                fdst(i0 + 2, dstv0, isemd0).start()
                fsrc(i0 + 2, srcv0, isems0).wait()
                gather(rows0, srcv0, gsem0).start()
                fsrc(i0 + 3, srcv1, isems1).start()
            fdst(i1, dstv1, isemd1).wait()
            sc1 = scatter(rows1, dstv1, ssem1)
            sc1.start(add=True)
            count(dstv1)
            if last:
                sc1.wait()

        pair_body(0, first=True)

        def pair(p, carry):
            pair_body(2 * p)
            return carry

        lax.fori_loop(1, NPAIR, pair, 0)
        pair_body(NCH - 2, last=True)
        plsc.subcore_barrier()

        r0 = s * RPT
        pltpu.sync_copy(acc.at[pl.ds(r0, RPT)], agg_out.at[c, pl.ds(r0, RPT)])
        if with_cnt:
            pltpu.sync_copy(cntv, cnt_out.at[wid])

    return body


_sc_agg_cnt_kernel = _sc_agg(with_cnt=True)
_sc_agg_kernel = _sc_agg(with_cnt=False)

BLK = 1024


def _tc_body(agg_ref, cnt_ref, x_ref, wl_ref, bl_ref, wr_ref, o_ref):
    agg = agg_ref[0] + agg_ref[1]
    cnt = lax.dot_general(cnt_ref[...], jnp.ones((NW, D), jnp.float32),
                          (((0,), (0,)), ((), ())),
                          preferred_element_type=jnp.float32)
    mean = agg / jnp.maximum(cnt, 1.0)
    acc = jnp.dot(mean, wl_ref[...], preferred_element_type=jnp.float32)
    acc = acc + bl_ref[...]
    acc = acc + jnp.dot(x_ref[...], wr_ref[...],
                        preferred_element_type=jnp.float32)
    o_ref[...] = jnp.maximum(acc, 0.0)


def _tc_layer(aggp, cntp, x, Wl, bl2d, Wr):
    return pl.pallas_call(
        _tc_body,
        grid=((N_NODES + BLK - 1) // BLK,),
        in_specs=[
            pl.BlockSpec((NC, BLK, D), lambda i: (0, i, 0)),
            pl.BlockSpec((NW, BLK), lambda i: (0, i)),
            pl.BlockSpec((BLK, D), lambda i: (i, 0)),
            pl.BlockSpec((D, D), lambda i: (0, 0)),
            pl.BlockSpec((1, D), lambda i: (0, 0)),
            pl.BlockSpec((D, D), lambda i: (0, 0)),
        ],
        out_specs=pl.BlockSpec((BLK, D), lambda i: (i, 0)),
        out_shape=jax.ShapeDtypeStruct((N_NODES, D), jnp.float32),
    )(aggp, cntp, x, Wl, bl2d, Wr)


def kernel(x, adj, Wl1, bl1, Wr1, Wl2, bl2, Wr2):
    adj = adj.astype(jnp.int32)
    npad_e = E_PAD - N_EDGES
    pad_src = jnp.arange(npad_e, dtype=jnp.int32) % N_NODES
    pad_dst = N_NODES + (jnp.arange(npad_e, dtype=jnp.int32) % (NPAD - N_NODES))
    src = jnp.concatenate([adj[0], pad_src])
    dst = jnp.concatenate([adj[1], pad_dst])
    zrow = jnp.zeros((RPT, D), jnp.float32)
    zcnt = jnp.zeros((NPAD,), jnp.float32)

    aggp, cntp = _sc_agg_cnt_kernel(x, src, dst, zrow, zcnt)
    h = _tc_layer(aggp, cntp, x, Wl1, bl1.reshape(1, D), Wr1)
    (aggp2,) = _sc_agg_kernel(h, src, dst, zrow)
    out = _tc_layer(aggp2, cntp, h, Wl2, bl2.reshape(1, D), Wr2)
    return out
